# baseline (speedup 1.0000x reference)
import functools

import jax
import jax.numpy as jnp
from jax import lax
from jax.experimental import pallas as pl
from jax.experimental.pallas import tpu as pltpu
from jax.experimental.pallas import tpu_sc as plsc

BS = 2048
D = 256
TK = 2048
NC, NS, L = 2, 16, 16
NW = NC * NS
BPW = BS // NW


def _nn_body(n_valid, n1, q2_ref, qn_ref, yn1_ref, yn2_ref, colf_ref,
             y1_ref, y2_ref, best1_ref, bidx1_ref, best2_ref, bidx2_ref):
    j = pl.program_id(0)
    is2 = j == n1
    y = jnp.where(is2, y2_ref[...], y1_ref[...])
    row = lax.broadcasted_iota(jnp.int32, (TK, 1), 0) + j * TK
    y = jnp.where(is2 | (row < n_valid), y, 0.0)
    yn = jnp.where(is2, yn2_ref[0], yn1_ref[0])
    qy2 = lax.dot_general(q2_ref[...], y, (((1,), (1,)), ((), ())),
                          preferred_element_type=jnp.float32)
    s = (qn_ref[...] + yn) + qy2
    colf = colf_ref[0]
    m = jnp.min(s, axis=1, keepdims=True)
    hit = s == m
    af = jnp.min(jnp.where(hit, colf, jnp.float32(3e38)), axis=1,
                 keepdims=True)
    a = af.astype(jnp.int32)

    @pl.when(jnp.logical_not(is2))
    def _():
        prev_m = jnp.where(j == 0, jnp.inf, best1_ref[...])
        prev_a = jnp.where(j == 0, 0, bidx1_ref[...])
        better = m < prev_m
        best1_ref[...] = jnp.where(better, m, prev_m)
        bidx1_ref[...] = jnp.where(better, a + j * TK, prev_a)

    @pl.when(is2)
    def _():
        best2_ref[...] = m
        bidx2_ref[...] = a


def _nn_searches(q2, qn, keys, yn_pad, n_valid, y2, yn2, interpret=False):
    n1 = yn_pad.shape[0] // TK
    yn3 = yn_pad.reshape(n1, 1, TK)
    yn2r = yn2.reshape(1, 1, TK)
    colf = jnp.arange(TK, dtype=jnp.float32).reshape(1, 1, TK)
    cl = n1 - 1
    b1, i1, b2, i2 = pl.pallas_call(
        functools.partial(_nn_body, n_valid, n1),
        grid=(n1 + 1,),
        in_specs=[
            pl.BlockSpec((BS, D), lambda j: (0, 0)),
            pl.BlockSpec((BS, 1), lambda j: (0, 0)),
            pl.BlockSpec((1, 1, TK), lambda j: (jnp.minimum(j, cl), 0, 0)),
            pl.BlockSpec((1, 1, TK), lambda j: (0, 0, 0)),
            pl.BlockSpec((1, 1, TK), lambda j: (0, 0, 0)),
            pl.BlockSpec((TK, D), lambda j: (jnp.minimum(j, cl), 0)),
            pl.BlockSpec((TK, D), lambda j: (0, 0)),
        ],
        out_specs=[
            pl.BlockSpec((BS, 1), lambda j: (0, 0)),
            pl.BlockSpec((BS, 1), lambda j: (0, 0)),
            pl.BlockSpec((BS, 1), lambda j: (0, 0)),
            pl.BlockSpec((BS, 1), lambda j: (0, 0)),
        ],
        out_shape=[
            jax.ShapeDtypeStruct((BS, 1), jnp.float32),
            jax.ShapeDtypeStruct((BS, 1), jnp.int32),
            jax.ShapeDtypeStruct((BS, 1), jnp.float32),
            jax.ShapeDtypeStruct((BS, 1), jnp.int32),
        ],
        interpret=interpret,
    )(q2, qn, yn3, yn2r, colf, keys, y2)
    return b1.reshape(BS), i1.reshape(BS), b2.reshape(BS), i2.reshape(BS)


def _sc_finish(table, idx_prev, idx_new, d_prev, d_new):
    mesh = plsc.VectorSubcoreMesh(core_axis_name="c", subcore_axis_name="s",
                                  num_cores=NC, num_subcores=NS)

    @functools.partial(
        pl.kernel,
        out_type=[
            jax.ShapeDtypeStruct((BS,), jnp.float32),
            jax.ShapeDtypeStruct((BS, D), jnp.float32),
        ],
        mesh=mesh,
        scratch_types=[
            pltpu.VMEM((BPW,), jnp.int32),
            pltpu.VMEM((BPW,), jnp.int32),
            pltpu.VMEM((BPW, D), jnp.float32),
            pltpu.VMEM((BPW, D), jnp.float32),
            pltpu.VMEM((BPW, D), jnp.float32),
            pltpu.VMEM((BPW,), jnp.float32),
            pltpu.VMEM((BPW,), jnp.float32),
            pltpu.VMEM((BPW,), jnp.float32),
            pltpu.SemaphoreType.DMA,
            pltpu.SemaphoreType.DMA,
        ],
    )
    def k(tab_hbm, ip_hbm, in_hbm, dp_hbm, dn_hbm,
          vals_hbm, grads_hbm,
          ip_v, in_v, p_v, n_v, g_v, dp_v, dn_v, v_v, sem1, sem2):
        wid = lax.axis_index("s") * NC + lax.axis_index("c")
        base = wid * BPW
        pltpu.sync_copy(ip_hbm.at[pl.ds(base, BPW)], ip_v)
        pltpu.sync_copy(in_hbm.at[pl.ds(base, BPW)], in_v)
        cp1 = pltpu.async_copy(tab_hbm.at[ip_v], p_v, sem1)
        cp2 = pltpu.async_copy(tab_hbm.at[in_v], n_v, sem2)
        pltpu.sync_copy(dp_hbm.at[pl.ds(base, BPW)], dp_v)
        pltpu.sync_copy(dn_hbm.at[pl.ds(base, BPW)], dn_v)
        for c in range(BPW // L):
            sl = pl.ds(c * L, L)
            v_v[sl] = dp_v[sl] - dn_v[sl]
        pltpu.sync_copy(v_v, vals_hbm.at[pl.ds(base, BPW)])
        cp1.wait()
        cp2.wait()

        def row(r, carry):
            for c in range(D // L):
                sl = pl.ds(c * L, L)
                g_v[r, sl] = (n_v[r, sl] - p_v[r, sl]) * 2.0
            return carry

        lax.fori_loop(0, BPW, row, 0)
        pltpu.sync_copy(g_v, grads_hbm.at[pl.ds(base, BPW)])

    return k(table, idx_prev, idx_new, d_prev, d_new)


def kernel(target_goal_embedding_library, reached_goal_embedding_library,
           batch_size):
    N = target_goal_embedding_library.shape[0]
    start = N - batch_size
    q = lax.dynamic_slice_in_dim(target_goal_embedding_library, start, BS)
    reached = lax.dynamic_slice_in_dim(reached_goal_embedding_library, start, BS)
    n_prev = N - BS

    qn = jnp.sum(q * q, axis=1, keepdims=True)
    q2 = -2.0 * q

    n_pad = -(-n_prev // TK) * TK
    yk = reached_goal_embedding_library[:n_pad]
    yn_prev = jnp.sum(yk * yk, axis=1)
    yn_prev = jnp.where(jnp.arange(n_pad) < n_prev, yn_prev, jnp.inf)
    yn_new = jnp.sum(reached * reached, axis=1)

    d_prev, idx_prev, d_new, idx_new = _nn_searches(
        q2, qn, reached_goal_embedding_library, yn_prev, n_prev,
        reached, yn_new)

    idx_new_abs = idx_new + jnp.int32(start)
    vals, grads = _sc_finish(reached_goal_embedding_library,
                             idx_prev, idx_new_abs, d_prev, d_new)
    return vals, grads


# speedup vs baseline: 1.0081x; 1.0081x over previous
import functools

import jax
import jax.numpy as jnp
from jax import lax
from jax.experimental import pallas as pl
from jax.experimental.pallas import tpu as pltpu
from jax.experimental.pallas import tpu_sc as plsc

BS = 2048
D = 256
TK = 2048
NC, NS, L = 2, 16, 16
NW = NC * NS
BPW = BS // NW


def _nn_body(n_valid, n_steps, q2_ref, qn_ref, yn_ref, colf_ref, y_ref,
             best_ref, bidx_ref):
    j = pl.program_id(0)
    y = y_ref[...]
    if n_valid % TK != 0:
        row = lax.broadcasted_iota(jnp.int32, (TK, 1), 0) + j * TK
        y = jnp.where(row < n_valid, y, 0.0)
    yn = yn_ref[0]
    qy2 = lax.dot_general(q2_ref[...], y, (((1,), (1,)), ((), ())),
                          preferred_element_type=jnp.float32)
    s = (qn_ref[...] + yn) + qy2
    colf = colf_ref[0]
    m = jnp.min(s, axis=1, keepdims=True)
    hit = s == m
    af = jnp.min(jnp.where(hit, colf, jnp.float32(3e38)), axis=1,
                 keepdims=True)
    a = af.astype(jnp.int32) + j * TK
    prev_m = jnp.where(j == 0, jnp.inf, best_ref[...])
    prev_a = jnp.where(j == 0, 0, bidx_ref[...])
    better = m < prev_m
    best_ref[...] = jnp.where(better, m, prev_m)
    bidx_ref[...] = jnp.where(better, a, prev_a)


def _nn_argmin(q2, qn, keys, yn_pad, n_valid, interpret=False):
    n_steps = yn_pad.shape[0] // TK
    yn3 = yn_pad.reshape(n_steps, 1, TK)
    colf = jnp.arange(TK, dtype=jnp.float32).reshape(1, 1, TK)
    best, bidx = pl.pallas_call(
        functools.partial(_nn_body, n_valid, n_steps),
        grid=(n_steps,),
        in_specs=[
            pl.BlockSpec((BS, D), lambda j: (0, 0)),
            pl.BlockSpec((BS, 1), lambda j: (0, 0)),
            pl.BlockSpec((1, 1, TK), lambda j: (j, 0, 0)),
            pl.BlockSpec((1, 1, TK), lambda j: (0, 0, 0)),
            pl.BlockSpec((TK, D), lambda j: (j, 0)),
        ],
        out_specs=[
            pl.BlockSpec((BS, 1), lambda j: (0, 0)),
            pl.BlockSpec((BS, 1), lambda j: (0, 0)),
        ],
        out_shape=[
            jax.ShapeDtypeStruct((BS, 1), jnp.float32),
            jax.ShapeDtypeStruct((BS, 1), jnp.int32),
        ],
        interpret=interpret,
    )(q2, qn, yn3, colf, keys)
    return best.reshape(BS), bidx.reshape(BS)


def _sc_finish(table, idx_prev, idx_new, d_prev, d_new):
    mesh = plsc.VectorSubcoreMesh(core_axis_name="c", subcore_axis_name="s",
                                  num_cores=NC, num_subcores=NS)

    @functools.partial(
        pl.kernel,
        out_type=[
            jax.ShapeDtypeStruct((BS,), jnp.float32),
            jax.ShapeDtypeStruct((BS, D), jnp.float32),
        ],
        mesh=mesh,
        scratch_types=[
            pltpu.VMEM((BPW,), jnp.int32),
            pltpu.VMEM((BPW,), jnp.int32),
            pltpu.VMEM((BPW, D), jnp.float32),
            pltpu.VMEM((BPW, D), jnp.float32),
            pltpu.VMEM((BPW, D), jnp.float32),
            pltpu.VMEM((BPW,), jnp.float32),
            pltpu.VMEM((BPW,), jnp.float32),
            pltpu.VMEM((BPW,), jnp.float32),
            pltpu.SemaphoreType.DMA,
            pltpu.SemaphoreType.DMA,
        ],
    )
    def k(tab_hbm, ip_hbm, in_hbm, dp_hbm, dn_hbm,
          vals_hbm, grads_hbm,
          ip_v, in_v, p_v, n_v, g_v, dp_v, dn_v, v_v, sem1, sem2):
        wid = lax.axis_index("s") * NC + lax.axis_index("c")
        base = wid * BPW
        pltpu.sync_copy(ip_hbm.at[pl.ds(base, BPW)], ip_v)
        pltpu.sync_copy(in_hbm.at[pl.ds(base, BPW)], in_v)
        cp1 = pltpu.async_copy(tab_hbm.at[ip_v], p_v, sem1)
        cp2 = pltpu.async_copy(tab_hbm.at[in_v], n_v, sem2)
        pltpu.sync_copy(dp_hbm.at[pl.ds(base, BPW)], dp_v)
        pltpu.sync_copy(dn_hbm.at[pl.ds(base, BPW)], dn_v)
        for c in range(BPW // L):
            sl = pl.ds(c * L, L)
            v_v[sl] = dp_v[sl] - dn_v[sl]
        pltpu.sync_copy(v_v, vals_hbm.at[pl.ds(base, BPW)])
        cp1.wait()
        cp2.wait()

        def row(r, carry):
            for c in range(D // L):
                sl = pl.ds(c * L, L)
                g_v[r, sl] = (n_v[r, sl] - p_v[r, sl]) * 2.0
            return carry

        lax.fori_loop(0, BPW, row, 0)
        pltpu.sync_copy(g_v, grads_hbm.at[pl.ds(base, BPW)])

    return k(table, idx_prev, idx_new, d_prev, d_new)


def kernel(target_goal_embedding_library, reached_goal_embedding_library,
           batch_size):
    N = target_goal_embedding_library.shape[0]
    start = N - batch_size
    q = lax.dynamic_slice_in_dim(target_goal_embedding_library, start, BS)
    reached = lax.dynamic_slice_in_dim(reached_goal_embedding_library, start, BS)
    n_prev = N - BS

    qn = jnp.sum(q * q, axis=1, keepdims=True)
    q2 = -2.0 * q

    n_pad = -(-n_prev // TK) * TK
    yk = reached_goal_embedding_library[:n_pad]
    yn_prev = jnp.sum(yk * yk, axis=1)
    yn_prev = jnp.where(jnp.arange(n_pad) < n_prev, yn_prev, jnp.inf)
    yn_new = jnp.sum(reached * reached, axis=1)

    d_prev, idx_prev = _nn_argmin(q2, qn, reached_goal_embedding_library,
                                  yn_prev, n_prev)
    d_new, idx_new = _nn_argmin(q2, qn, reached, yn_new, BS)

    idx_new_abs = idx_new + jnp.int32(start)
    vals, grads = _sc_finish(reached_goal_embedding_library,
                             idx_prev, idx_new_abs, d_prev, d_new)
    return vals, grads
